# baseline (speedup 1.0000x reference)
import jax
import jax.numpy as jnp
from jax import lax
from jax.experimental import pallas as pl
from jax.experimental.pallas import tpu as pltpu
from jax.experimental.pallas import tpu_sc as plsc

_B = 128
_N = 32768
_NW32 = _N // 4

_NC = 2
_NS = 16
_NWA = 16
_RPW = _B // _NWA
_CW = 128
_L = 16
_NG = _CW // _L
_BIGF = 1 << 30


def _scan_chunk(load16):
    best = jnp.int32(_BIGF)
    lanes = lax.iota(jnp.int32, _L)
    for g in range(_NG):
        w = load16(g)
        nz = w != 0
        b0 = (w & 0x000000FF) != 0
        b1 = (w & 0x0000FF00) != 0
        b2 = (w & 0x00FF0000) != 0
        byte_idx = jnp.where(b0, 0, jnp.where(b1, 1, jnp.where(b2, 2, 3)))
        cand = (g * _L + lanes) * 4 + byte_idx
        best = jnp.minimum(best, jnp.min(jnp.where(nz, cand, _BIGF)))
    return best


def _sc_body(mask_hbm, fv_hbm, buf_ref, spare_ref, res_ref, sem):
    c = lax.axis_index("c")
    s = lax.axis_index("s")
    wid = s * _NC + c

    @pl.when(wid < _NWA)
    def _active():
        base_row = wid * _RPW
        copies = [
            pltpu.make_async_copy(
                mask_hbm.at[pl.ds(
                    pl.multiple_of((base_row + r) * _NW32, _CW), _CW)],
                buf_ref.at[pl.ds(r * _CW, _CW)], sem)
            for r in range(_RPW)
        ]
        for cp in copies:
            cp.start()
        for cp in copies:
            cp.wait()

        for r in range(_RPW):
            row = base_row + r
            best0 = _scan_chunk(
                lambda g: buf_ref[pl.ds(r * _CW + g * _L, _L)])
            found0 = jnp.where(best0 < _BIGF, best0, jnp.int32(-1))

            def outer_cond(carry):
                off, fnd = carry
                return jnp.logical_and(fnd < 0, off < _NW32)

            def outer_body(carry):
                off, fnd = carry
                pltpu.sync_copy(
                    mask_hbm.at[pl.ds(
                        pl.multiple_of(row * _NW32 + off, _CW), _CW)],
                    spare_ref)
                best = _scan_chunk(lambda g: spare_ref[pl.ds(g * _L, _L)])
                fnd = jnp.where(best < _BIGF, off * 4 + best, fnd)
                return (off + jnp.int32(_CW), fnd)

            _, found = lax.while_loop(
                outer_cond, outer_body, (jnp.int32(_CW), found0))

            res_ref[r, :] = jnp.broadcast_to(found, (_L,)).astype(jnp.int32)

        pltpu.sync_copy(res_ref, fv_hbm.at[pl.ds(base_row, _RPW)])


def _first_valid_sc(mask_words_flat):
    mesh = plsc.VectorSubcoreMesh(core_axis_name="c", subcore_axis_name="s")
    return pl.kernel(
        _sc_body,
        out_type=jax.ShapeDtypeStruct((_B, _L), jnp.int32),
        mesh=mesh,
        scratch_types=[
            pltpu.VMEM((_RPW * _CW,), jnp.int32),
            pltpu.VMEM((_CW,), jnp.int32),
            pltpu.VMEM((_RPW, _L), jnp.int32),
            pltpu.SemaphoreType.DMA,
        ],
        compiler_params=pltpu.CompilerParams(needs_layout_passes=False),
    )(mask_words_flat)


_BN = 2048
_NBLK = _N // _BN


def _tc_body(fv_ref, out_ref):
    j = pl.program_id(0)
    fvv = jnp.min(fv_ref[...], axis=1, keepdims=True)
    iota = lax.broadcasted_iota(jnp.int32, (_B, _BN), 1) + j * _BN
    out_ref[...] = jnp.where(iota == fvv, jnp.float32(10.0), jnp.float32(0.0))


def _write_logits_tc(fv):
    return pl.pallas_call(
        _tc_body,
        grid=(_NBLK,),
        in_specs=[pl.BlockSpec((_B, _L), lambda j: (0, 0))],
        out_specs=pl.BlockSpec((_B, _BN), lambda j: (0, j)),
        out_shape=jax.ShapeDtypeStruct((_B, _N), jnp.float32),
        compiler_params=pltpu.CompilerParams(
            dimension_semantics=("arbitrary",),
        ),
    )(fv)


def kernel(action_mask):
    m8 = action_mask.astype(jnp.uint8).reshape(_B, _N // 4, 4)
    m32 = lax.bitcast_convert_type(m8, jnp.int32).reshape(-1)
    fv = _first_valid_sc(m32)
    return _write_logits_tc(fv)
